# baseline (speedup 1.0000x reference)
import functools

import jax
import jax.numpy as jnp
from jax import lax
from jax.experimental import pallas as pl
from jax.experimental.pallas import tpu as pltpu
from jax.experimental.pallas import tpu_sc as plsc


@functools.lru_cache(maxsize=None)
def _make_gather(b: int, l: int, d: int, bc: int):
    info = plsc.get_sparse_core_info()
    nc, ns = info.num_cores, info.num_subcores
    nw = nc * ns
    nbc = b // bc
    units = l * nbc
    iters = (units + nw - 1) // nw
    mesh = plsc.VectorSubcoreMesh(core_axis_name="c", subcore_axis_name="s")

    @functools.partial(
        pl.kernel,
        mesh=mesh,
        out_type=jax.ShapeDtypeStruct((l, b, d), jnp.float32),
        scratch_types=[
            pltpu.VMEM((l, nbc, bc), jnp.int32),
            pltpu.VMEM((bc, d), jnp.float32),
            pltpu.SemaphoreType.DMA,
        ],
        compiler_params=pltpu.CompilerParams(use_tc_tiling_on_sc=True),
    )
    def gather(table_hbm, idx_hbm, out_hbm, idx_v, rows_v, sem):
        wid = lax.axis_index("s") * nc + lax.axis_index("c")
        pltpu.sync_copy(idx_hbm, idx_v)

        def body(k, carry):
            g = wid + nw * k

            @pl.when(g < units)
            def _():
                li = g // nbc
                ci = g % nbc
                pltpu.async_copy(
                    table_hbm.at[idx_v.at[li].at[ci]], rows_v, sem
                ).wait()
                pltpu.sync_copy(rows_v, out_hbm.at[li].at[pl.ds(ci * bc, bc)])

            return carry

        lax.fori_loop(0, iters, body, 0)

    return gather


def _mlp_body(emb_ref, w1t_ref, b1_ref, w2t_ref, b2_ref, out_ref):
    embt = emb_ref[0].T.astype(jnp.bfloat16)
    h = jnp.dot(
        w1t_ref[...].astype(jnp.bfloat16), embt,
        preferred_element_type=jnp.float32,
    )
    h = jnp.maximum(h + b1_ref[...], 0.0)
    out_ref[0] = (
        jnp.dot(
            w2t_ref[...].astype(jnp.bfloat16), h.astype(jnp.bfloat16),
            preferred_element_type=jnp.float32,
        )
        + b2_ref[...]
    )


@functools.lru_cache(maxsize=None)
def _make_mlp(b: int, l: int, d: int, hidden: int, vocab: int):
    return pl.pallas_call(
        _mlp_body,
        grid=(l,),
        in_specs=[
            pl.BlockSpec((1, b, d), lambda i: (i, 0, 0)),
            pl.BlockSpec((hidden, d), lambda i: (0, 0)),
            pl.BlockSpec((hidden, 1), lambda i: (0, 0)),
            pl.BlockSpec((vocab, hidden), lambda i: (0, 0)),
            pl.BlockSpec((vocab, 1), lambda i: (0, 0)),
        ],
        out_specs=pl.BlockSpec((1, vocab, b), lambda i: (i, 0, 0)),
        out_shape=jax.ShapeDtypeStruct((l, vocab, b), jnp.float32),
        compiler_params=pltpu.CompilerParams(
            dimension_semantics=("parallel",),
        ),
    )


def kernel(input_ids, table, W1, b1, W2, b2):
    b, l = input_ids.shape
    vocab, d = table.shape
    hidden = W1.shape[1]
    bc = 128

    idsT3 = jnp.clip(
        input_ids.astype(jnp.int32).T.reshape(l, b // bc, bc), 0, vocab - 1
    )
    embT = _make_gather(b, l, d, bc)(table, idsT3)

    logitsT = _make_mlp(b, l, d, hidden, vocab)(
        embT,
        W1.T,
        b1.reshape(hidden, 1),
        W2.T,
        b2.reshape(vocab, 1),
    )
    return jnp.transpose(logitsT, (2, 0, 1))


# speedup vs baseline: 1.0117x; 1.0117x over previous
import functools

import jax
import jax.numpy as jnp
from jax import lax
from jax.experimental import pallas as pl
from jax.experimental.pallas import tpu as pltpu
from jax.experimental.pallas import tpu_sc as plsc


@functools.lru_cache(maxsize=None)
def _make_gather(b: int, l: int, d: int, bc: int):
    info = plsc.get_sparse_core_info()
    nc, ns = info.num_cores, info.num_subcores
    nw = nc * ns
    nbc = b // bc
    units = l * nbc
    iters = (units + nw - 1) // nw
    mesh = plsc.VectorSubcoreMesh(core_axis_name="c", subcore_axis_name="s")

    @functools.partial(
        pl.kernel,
        mesh=mesh,
        out_type=jax.ShapeDtypeStruct((l, b, d), jnp.float32),
        scratch_types=[
            pltpu.VMEM((l, nbc, bc), jnp.int32),
            pltpu.VMEM((2, bc, d), jnp.float32),
            pltpu.SemaphoreType.DMA,
            pltpu.SemaphoreType.DMA,
        ],
        compiler_params=pltpu.CompilerParams(use_tc_tiling_on_sc=True),
    )
    def gather(table_hbm, idx_hbm, out_hbm, idx_v, rows_v, sem0, sem1):
        wid = lax.axis_index("s") * nc + lax.axis_index("c")
        pltpu.sync_copy(idx_hbm, idx_v)
        sems = (sem0, sem1)
        full, tail = divmod(units, nw)

        def unit(k):
            g = wid + nw * k
            return g // nbc, g % nbc

        def guarded(k, fn):
            if k < full:
                fn()
            else:
                @pl.when(wid < tail)
                def _():
                    fn()

        def issue(k):
            li, ci = unit(k)
            pltpu.async_copy(
                table_hbm.at[idx_v.at[li].at[ci]], rows_v.at[k % 2], sems[k % 2]
            )

        def finish(k):
            li, ci = unit(k)
            pltpu.make_async_copy(
                table_hbm.at[idx_v.at[li].at[ci]], rows_v.at[k % 2], sems[k % 2]
            ).wait()
            pltpu.sync_copy(
                rows_v.at[k % 2], out_hbm.at[li].at[pl.ds(ci * bc, bc)]
            )

        guarded(0, lambda: issue(0))
        for k in range(iters):
            if k + 1 < iters:
                guarded(k + 1, lambda k=k: issue(k + 1))
            guarded(k, lambda k=k: finish(k))

    return gather


def _mlp_body(emb_ref, w1t_ref, b1_ref, w2t_ref, b2_ref, out_ref):
    embt = emb_ref[0].T.astype(jnp.bfloat16)
    h = jnp.dot(
        w1t_ref[...].astype(jnp.bfloat16), embt,
        preferred_element_type=jnp.float32,
    )
    h = jnp.maximum(h + b1_ref[...], 0.0)
    out_ref[0] = (
        jnp.dot(
            w2t_ref[...].astype(jnp.bfloat16), h.astype(jnp.bfloat16),
            preferred_element_type=jnp.float32,
        )
        + b2_ref[...]
    )


@functools.lru_cache(maxsize=None)
def _make_mlp(b: int, l: int, d: int, hidden: int, vocab: int):
    return pl.pallas_call(
        _mlp_body,
        grid=(l,),
        in_specs=[
            pl.BlockSpec((1, b, d), lambda i: (i, 0, 0)),
            pl.BlockSpec((hidden, d), lambda i: (0, 0)),
            pl.BlockSpec((hidden, 1), lambda i: (0, 0)),
            pl.BlockSpec((vocab, hidden), lambda i: (0, 0)),
            pl.BlockSpec((vocab, 1), lambda i: (0, 0)),
        ],
        out_specs=pl.BlockSpec((1, vocab, b), lambda i: (i, 0, 0)),
        out_shape=jax.ShapeDtypeStruct((l, vocab, b), jnp.float32),
        compiler_params=pltpu.CompilerParams(
            dimension_semantics=("parallel",),
        ),
    )


def kernel(input_ids, table, W1, b1, W2, b2):
    b, l = input_ids.shape
    vocab, d = table.shape
    hidden = W1.shape[1]
    bc = 128

    idsT3 = jnp.clip(
        input_ids.astype(jnp.int32).T.reshape(l, b // bc, bc), 0, vocab - 1
    )
    embT = _make_gather(b, l, d, bc)(table, idsT3)

    logitsT = _make_mlp(b, l, d, hidden, vocab)(
        embT,
        W1.T,
        b1.reshape(hidden, 1),
        W2.T,
        b2.reshape(vocab, 1),
    )
    return jnp.transpose(logitsT, (2, 0, 1))


# speedup vs baseline: 1.0663x; 1.0540x over previous
import functools

import jax
import jax.numpy as jnp
from jax import lax
from jax.experimental import pallas as pl
from jax.experimental.pallas import tpu as pltpu
from jax.experimental.pallas import tpu_sc as plsc


@functools.lru_cache(maxsize=None)
def _make_gather(b: int, l: int, d: int, bc: int):
    info = plsc.get_sparse_core_info()
    nc, ns = info.num_cores, info.num_subcores
    nw = nc * ns
    nbc = b // bc
    units = l * nbc
    iters = (units + nw - 1) // nw
    mesh = plsc.VectorSubcoreMesh(core_axis_name="c", subcore_axis_name="s")

    @functools.partial(
        pl.kernel,
        mesh=mesh,
        out_type=jax.ShapeDtypeStruct((l, b, d), jnp.float32),
        scratch_types=[
            pltpu.VMEM((iters // nbc + 2, nbc, bc), jnp.int32),
            pltpu.VMEM((3, bc, d), jnp.float32),
            pltpu.SemaphoreType.DMA,
            pltpu.SemaphoreType.DMA,
            pltpu.SemaphoreType.DMA,
            pltpu.SemaphoreType.DMA,
            pltpu.SemaphoreType.DMA,
            pltpu.SemaphoreType.DMA,
        ],
        compiler_params=pltpu.CompilerParams(use_tc_tiling_on_sc=True),
    )
    def gather(table_hbm, idx_hbm, out_hbm, idx_v, rows_v, g0, g1, g2, w0, w1, w2):
        wid = lax.axis_index("s") * nc + lax.axis_index("c")
        full, tail = divmod(units, nw)
        start = full * wid + jnp.minimum(wid, tail)
        n_l = (full + 1 + nbc - 1) // nbc + 1
        l_base = jnp.minimum(start // nbc, l - n_l)
        pltpu.sync_copy(idx_hbm.at[pl.ds(l_base, n_l)], idx_v)
        gsems = (g0, g1, g2)
        wsems = (w0, w1, w2)

        def unit(k):
            u = start + k
            return u // nbc - l_base, u % nbc, u // nbc, u % nbc

        def guarded(k, fn):
            if k < full:
                fn()
            else:
                @pl.when(wid < tail)
                def _():
                    fn()

        def issue(k):
            lv, cv, _, _ = unit(k)
            pltpu.async_copy(
                table_hbm.at[idx_v.at[lv].at[cv]], rows_v.at[k % 3], gsems[k % 3]
            )

        def finish(k):
            lv, cv, lo, co = unit(k)
            pltpu.make_async_copy(
                table_hbm.at[idx_v.at[lv].at[cv]], rows_v.at[k % 3], gsems[k % 3]
            ).wait()
            pltpu.async_copy(
                rows_v.at[k % 3], out_hbm.at[lo].at[pl.ds(co * bc, bc)], wsems[k % 3]
            )

        def drain_write(k):
            lv, cv, lo, co = unit(k)
            pltpu.make_async_copy(
                rows_v.at[k % 3], out_hbm.at[lo].at[pl.ds(co * bc, bc)], wsems[k % 3]
            ).wait()

        guarded(0, lambda: issue(0))
        if iters > 1:
            guarded(1, lambda: issue(1))
        for k in range(iters):
            if k + 2 < iters:
                if k - 1 >= 0:
                    guarded(k - 1, lambda k=k: drain_write(k - 1))
                guarded(k + 2, lambda k=k: issue(k + 2))
            guarded(k, lambda k=k: finish(k))
        for k in range(max(iters - 3, 0), iters):
            if k >= 0:
                guarded(k, lambda k=k: drain_write(k))

    return gather


def _mlp_body(emb_ref, w1t_ref, b1_ref, w2t_ref, b2_ref, out_ref):
    embt = emb_ref[0].T.astype(jnp.bfloat16)
    h = jnp.dot(
        w1t_ref[...].astype(jnp.bfloat16), embt,
        preferred_element_type=jnp.float32,
    )
    h = jnp.maximum(h + b1_ref[...], 0.0)
    out_ref[0] = (
        jnp.dot(
            w2t_ref[...].astype(jnp.bfloat16), h.astype(jnp.bfloat16),
            preferred_element_type=jnp.float32,
        )
        + b2_ref[...]
    )


@functools.lru_cache(maxsize=None)
def _make_mlp(b: int, l: int, d: int, hidden: int, vocab: int):
    return pl.pallas_call(
        _mlp_body,
        grid=(l,),
        in_specs=[
            pl.BlockSpec((1, b, d), lambda i: (i, 0, 0)),
            pl.BlockSpec((hidden, d), lambda i: (0, 0)),
            pl.BlockSpec((hidden, 1), lambda i: (0, 0)),
            pl.BlockSpec((vocab, hidden), lambda i: (0, 0)),
            pl.BlockSpec((vocab, 1), lambda i: (0, 0)),
        ],
        out_specs=pl.BlockSpec((1, vocab, b), lambda i: (i, 0, 0)),
        out_shape=jax.ShapeDtypeStruct((l, vocab, b), jnp.float32),
        compiler_params=pltpu.CompilerParams(
            dimension_semantics=("parallel",),
        ),
    )


def kernel(input_ids, table, W1, b1, W2, b2):
    b, l = input_ids.shape
    vocab, d = table.shape
    hidden = W1.shape[1]
    bc = 128

    idsT3 = jnp.clip(
        input_ids.astype(jnp.int32).T.reshape(l, b // bc, bc), 0, vocab - 1
    )
    embT = _make_gather(b, l, d, bc)(table, idsT3)

    logitsT = _make_mlp(b, l, d, hidden, vocab)(
        embT,
        W1.T,
        b1.reshape(hidden, 1),
        W2.T,
        b2.reshape(vocab, 1),
    )
    return jnp.transpose(logitsT, (2, 0, 1))


# speedup vs baseline: 1.0823x; 1.0150x over previous
import functools

import jax
import jax.numpy as jnp
from jax import lax
from jax.experimental import pallas as pl
from jax.experimental.pallas import tpu as pltpu
from jax.experimental.pallas import tpu_sc as plsc


@functools.lru_cache(maxsize=None)
def _make_gather(b: int, l: int, d: int, bc: int):
    info = plsc.get_sparse_core_info()
    nc, ns = info.num_cores, info.num_subcores
    nw = nc * ns
    nbc = b // bc
    units = l * nbc
    iters = (units + nw - 1) // nw
    mesh = plsc.VectorSubcoreMesh(core_axis_name="c", subcore_axis_name="s")

    @functools.partial(
        pl.kernel,
        mesh=mesh,
        out_type=jax.ShapeDtypeStruct((l, b, d), jnp.float32),
        scratch_types=[
            pltpu.VMEM((iters // nbc + 2, nbc, bc), jnp.int32),
            pltpu.VMEM((3, bc, d), jnp.float32),
            pltpu.SemaphoreType.DMA,
            pltpu.SemaphoreType.DMA,
            pltpu.SemaphoreType.DMA,
            pltpu.SemaphoreType.DMA,
            pltpu.SemaphoreType.DMA,
            pltpu.SemaphoreType.DMA,
        ],
        compiler_params=pltpu.CompilerParams(use_tc_tiling_on_sc=True),
    )
    def gather(table_hbm, idx_hbm, out_hbm, idx_v, rows_v, g0, g1, g2, w0, w1, w2):
        wid = lax.axis_index("s") * nc + lax.axis_index("c")
        full, tail = divmod(units, nw)
        start = full * wid + jnp.minimum(wid, tail)
        n_l = (full + 1 + nbc - 1) // nbc + 1
        l_base = jnp.minimum(start // nbc, l - n_l)
        pltpu.sync_copy(idx_hbm.at[pl.ds(l_base, n_l)], idx_v)
        gsems = (g0, g1, g2)
        wsems = (w0, w1, w2)

        def unit(k):
            u = start + k
            return u // nbc - l_base, u % nbc, u // nbc, u % nbc

        def guarded(k, fn):
            if k < full:
                fn()
            else:
                @pl.when(wid < tail)
                def _():
                    fn()

        def issue(k):
            lv, cv, _, _ = unit(k)
            pltpu.async_copy(
                table_hbm.at[idx_v.at[lv].at[cv]], rows_v.at[k % 3], gsems[k % 3]
            )

        def finish(k):
            lv, cv, lo, co = unit(k)
            pltpu.make_async_copy(
                table_hbm.at[idx_v.at[lv].at[cv]], rows_v.at[k % 3], gsems[k % 3]
            ).wait()
            pltpu.async_copy(
                rows_v.at[k % 3], out_hbm.at[lo].at[pl.ds(co * bc, bc)], wsems[k % 3]
            )

        def drain_write(k):
            lv, cv, lo, co = unit(k)
            pltpu.make_async_copy(
                rows_v.at[k % 3], out_hbm.at[lo].at[pl.ds(co * bc, bc)], wsems[k % 3]
            ).wait()

        guarded(0, lambda: issue(0))
        if iters > 1:
            guarded(1, lambda: issue(1))
        for k in range(iters):
            if k + 2 < iters:
                if k - 1 >= 0:
                    guarded(k - 1, lambda k=k: drain_write(k - 1))
                guarded(k + 2, lambda k=k: issue(k + 2))
            guarded(k, lambda k=k: finish(k))
        for k in range(max(iters - 3, 0), iters):
            if k >= 0:
                guarded(k, lambda k=k: drain_write(k))

    return gather


def _mlp_body(emb_ref, w1t_ref, b1_ref, w2t_ref, b2_ref, out_ref):
    embt = emb_ref[0].T.astype(jnp.bfloat16)
    h = jnp.dot(
        w1t_ref[...].astype(jnp.bfloat16), embt,
        preferred_element_type=jnp.float32,
    )
    h = jnp.maximum(h + b1_ref[...], 0.0)
    out_ref[0] = (
        jnp.dot(
            w2t_ref[...].astype(jnp.bfloat16), h.astype(jnp.bfloat16),
            preferred_element_type=jnp.float32,
        )
        + b2_ref[...]
    )


@functools.lru_cache(maxsize=None)
def _make_mlp(b: int, l: int, d: int, hidden: int, vocab: int,
              lc: int, l0: int, aliased: bool):
    def body(*refs):
        _mlp_body(*refs[-6:])

    in_specs = [
        pl.BlockSpec((1, b, d), lambda i: (i, 0, 0)),
        pl.BlockSpec((hidden, d), lambda i: (0, 0)),
        pl.BlockSpec((hidden, 1), lambda i: (0, 0)),
        pl.BlockSpec((vocab, hidden), lambda i: (0, 0)),
        pl.BlockSpec((vocab, 1), lambda i: (0, 0)),
    ]
    kwargs = {}
    if aliased:
        in_specs = [pl.BlockSpec(memory_space=pl.ANY)] + in_specs
        kwargs["input_output_aliases"] = {0: 0}
    return pl.pallas_call(
        body if aliased else _mlp_body,
        grid=(lc,),
        in_specs=in_specs,
        out_specs=pl.BlockSpec((1, vocab, b), lambda i: (i + l0, 0, 0)),
        out_shape=jax.ShapeDtypeStruct((l, vocab, b), jnp.float32),
        compiler_params=pltpu.CompilerParams(
            dimension_semantics=("parallel",),
        ),
        **kwargs,
    )


def kernel(input_ids, table, W1, b1, W2, b2):
    b, l = input_ids.shape
    vocab, d = table.shape
    hidden = W1.shape[1]
    bc = 128

    idsT3 = jnp.clip(
        input_ids.astype(jnp.int32).T.reshape(l, b // bc, bc), 0, vocab - 1
    )
    w1t = W1.T
    b1c = b1.reshape(hidden, 1)
    w2t = W2.T
    b2c = b2.reshape(vocab, 1)

    lc = l // 2
    emb_lo = _make_gather(b, lc, d, bc)(table, idsT3[:lc])
    emb_hi = _make_gather(b, l - lc, d, bc)(table, idsT3[lc:])

    logits_lo = _make_mlp(b, l, d, hidden, vocab, lc, 0, False)(
        emb_lo, w1t, b1c, w2t, b2c
    )
    logitsT = _make_mlp(b, l, d, hidden, vocab, l - lc, lc, True)(
        logits_lo, emb_hi, w1t, b1c, w2t, b2c
    )
    return jnp.transpose(logitsT, (2, 0, 1))
